# baseline (speedup 1.0000x reference)
import functools

import jax
import jax.numpy as jnp
from jax import lax
from jax.experimental import pallas as pl
from jax.experimental.pallas import tpu as pltpu
from jax.experimental.pallas import tpu_sc as plsc

NEG = -1000000000.0
LN2 = 0.6931471805599453
SQRT2 = 1.4142135623730951


@functools.lru_cache(maxsize=None)
def _build(B, V):
    info = plsc.get_sparse_core_info()
    NC, NS = info.num_cores, info.num_subcores
    NW = NC * NS
    RPW = B // NW
    CH = 9088
    NCH = 11
    TAIL = V - NCH * CH
    NT = RPW * NCH
    NV = CH // 16
    NVM = (NV // 4) * 4

    mesh = plsc.VectorSubcoreMesh(core_axis_name="c", subcore_axis_name="s")

    @functools.partial(
        pl.kernel,
        out_type=jax.ShapeDtypeStruct((NW * 16,), jnp.float32),
        mesh=mesh,
        compiler_params=pltpu.CompilerParams(
            needs_layout_passes=False, use_tc_tiling_on_sc=True),
        scratch_types=[
            pltpu.VMEM((CH,), jnp.float32),
            pltpu.VMEM((CH,), jnp.float32),
            pltpu.VMEM((CH,), jnp.int32),
            pltpu.VMEM((CH,), jnp.int32),
            pltpu.VMEM((B,), jnp.int32),
            pltpu.VMEM((128,), jnp.float32),
            pltpu.VMEM((128,), jnp.int32),
            pltpu.VMEM((32,), jnp.float32),
            pltpu.VMEM((32,), jnp.int32),
            pltpu.VMEM((16,), jnp.float32),
            pltpu.SemaphoreType.DMA,
            pltpu.SemaphoreType.DMA,
            pltpu.SemaphoreType.DMA,
            pltpu.SemaphoreType.DMA,
        ],
    )
    def body(logits_hbm, mask_hbm, value_hbm, out_hbm,
             lb0, lb1, mb0, mb1, vbuf, gbuf, gmb, tb, tmb, obuf,
             semL0, semL1, semM0, semM1):
        wid = lax.axis_index("s") * NC + lax.axis_index("c")
        iota = lax.iota(jnp.int32, 16)
        negv = jnp.full((16,), NEG, jnp.float32)
        zerov = jnp.zeros((16,), jnp.float32)

        def rowci(t):
            return wid * RPW + t // NCH, t % NCH

        def start(t, lb, mb, semL, semM):
            row, ci = rowci(t)
            c0 = pl.multiple_of(ci * CH, 128)
            pltpu.async_copy(logits_hbm.at[row, pl.ds(c0, CH)], lb, semL)
            pltpu.async_copy(mask_hbm.at[row, pl.ds(c0, CH)], mb, semM)

        def wait(lb, mb, semL, semM):
            pltpu.make_async_copy(
                logits_hbm.at[0, pl.ds(0, CH)], lb, semL).wait()
            pltpu.make_async_copy(
                mask_hbm.at[0, pl.ds(0, CH)], mb, semM).wait()

        def masked_exp(x, mk, newm):
            xm = jnp.where(mk != 0, x, negv)
            return jnp.exp(xm - newm)

        def process(t, lb, mb, carry):
            Mv, Sv, m, s = carry
            row, ci = rowci(t)
            r = t // NCH
            first = ci == 0
            m = jnp.where(first, negv, m)
            s = jnp.where(first, zerov, s)

            @plsc.parallel_loop(0, NVM, step=4, unroll=8,
                                carry=(negv, negv, negv, negv))
            def p1(v, c):
                b = v * 16
                return (jnp.maximum(c[0], lb[pl.ds(b, 16)]),
                        jnp.maximum(c[1], lb[pl.ds(b + 16, 16)]),
                        jnp.maximum(c[2], lb[pl.ds(b + 32, 16)]),
                        jnp.maximum(c[3], lb[pl.ds(b + 48, 16)]))

            cmax = jnp.maximum(jnp.maximum(p1[0], p1[1]),
                               jnp.maximum(p1[2], p1[3]))
            @pl.when(ci == NCH - 1)
            def _():
                c0t = pl.multiple_of(NCH * CH, 128)
                pltpu.sync_copy(logits_hbm.at[row, pl.ds(c0t, TAIL)], tb)
                pltpu.sync_copy(mask_hbm.at[row, pl.ds(c0t, TAIL)], tmb)

            last = ci == NCH - 1
            tmax = jnp.maximum(tb[pl.ds(0, 16)], tb[pl.ds(16, 16)])
            cmax = jnp.where(last, jnp.maximum(cmax, tmax), cmax)
            newm = jnp.maximum(m, cmax)
            s = s * jnp.exp(m - newm)

            @plsc.parallel_loop(0, NVM, step=4, unroll=4,
                                carry=(zerov, zerov, zerov, zerov))
            def p2(v, a):
                b = v * 16
                return (
                    a[0] + masked_exp(lb[pl.ds(b, 16)],
                                      mb[pl.ds(b, 16)], newm),
                    a[1] + masked_exp(lb[pl.ds(b + 16, 16)],
                                      mb[pl.ds(b + 16, 16)], newm),
                    a[2] + masked_exp(lb[pl.ds(b + 32, 16)],
                                      mb[pl.ds(b + 32, 16)], newm),
                    a[3] + masked_exp(lb[pl.ds(b + 48, 16)],
                                      mb[pl.ds(b + 48, 16)], newm))

            s = s + ((p2[0] + p2[1]) + (p2[2] + p2[3]))
            tsum = (masked_exp(tb[pl.ds(0, 16)], tmb[pl.ds(0, 16)], newm)
                    + masked_exp(tb[pl.ds(16, 16)], tmb[pl.ds(16, 16)], newm))
            s = s + jnp.where(last, tsum, zerov)

            M = jnp.max(newm)
            Sg = jnp.sum(s * jnp.exp(newm - M))
            sel = last & (iota == r)
            Mv = jnp.where(sel, M, Mv)
            Sv = jnp.where(sel, Sg, Sv)
            return (Mv, Sv, newm, s)

        start(0, lb0, mb0, semL0, semM0)
        start(1, lb1, mb1, semL1, semM1)

        def loop_body(i, carry):
            t0 = 2 * i
            wait(lb0, mb0, semL0, semM0)
            carry = process(t0, lb0, mb0, carry)

            @pl.when(i < NT // 2 - 1)
            def _():
                start(t0 + 2, lb0, mb0, semL0, semM0)

            wait(lb1, mb1, semL1, semM1)
            carry = process(t0 + 1, lb1, mb1, carry)

            @pl.when(i < NT // 2 - 1)
            def _():
                start(t0 + 3, lb1, mb1, semL1, semM1)

            return carry

        Mv, Sv, _, _ = lax.fori_loop(
            0, NT // 2, loop_body,
            (zerov, jnp.ones((16,), jnp.float32), negv, zerov))

        pltpu.sync_copy(value_hbm, vbuf)
        vals = plsc.load_gather(vbuf, [wid * RPW + jnp.minimum(iota, RPW - 1)])
        Gv = negv
        for r in range(RPW):
            row = wid * RPW + r
            val = jnp.max(jnp.where(iota == r, vals, 0))
            va = pl.multiple_of((val // 128) * 128, 128)
            pltpu.sync_copy(logits_hbm.at[row, pl.ds(va, 128)], gbuf)
            pltpu.sync_copy(mask_hbm.at[row, pl.ds(va, 128)], gmb)
            off = val - va
            voff = (off // 16) * 16
            xv = gbuf[pl.ds(voff, 16)]
            mkv = gmb[pl.ds(voff, 16)]
            lane = off - voff
            hit = iota == lane
            g = jnp.max(jnp.where(hit & (mkv != 0), xv, negv))
            Gv = jnp.where(iota == r, g, Gv)

        bits = plsc.bitcast(Sv, jnp.int32)
        e = (lax.shift_right_logical(bits, 23) & 0xFF) - 127
        mant = plsc.bitcast((bits & 0x7FFFFF) | 0x3F800000, jnp.float32)
        big = mant > SQRT2
        mant = jnp.where(big, mant * 0.5, mant)
        e = jnp.where(big, e + 1, e)
        t = (mant - 1.0) / (mant + 1.0)
        t2 = t * t
        logm = 2.0 * t * (1.0 + t2 * (1.0 / 3.0 + t2 * (0.2 + t2 * (1.0 / 7.0))))
        logS = e.astype(jnp.float32) * LN2 + logm

        obuf[...] = Gv - (Mv + logS)
        pltpu.sync_copy(obuf, out_hbm.at[pl.ds(wid * 16, 16)])

    return body, RPW


def kernel(logits, mask, value):
    B, V = logits.shape
    body, rpw = _build(B, V)
    out = body(logits, mask, value.astype(jnp.int32))
    return out.reshape(B // rpw, 16)[:, :rpw].reshape(B)
